# Initial kernel scaffold; baseline (speedup 1.0000x reference)
import jax
import jax.numpy as jnp
from jax.experimental import pallas as pl


def kernel(pred, gt, pcd_radius):
    raise NotImplementedError("write your pallas kernel here")


# speedup vs baseline: 14.6128x; 14.6128x over previous
import math

import jax
import jax.numpy as jnp
from jax.experimental import pallas as pl

B = 8
N = 2048
NPOINT = int(N * 0.05)
PCTS = (0.004, 0.006, 0.008, 0.01, 0.012)
NSAMPLES = tuple(int(N * p) for p in PCTS)
EXPECT_LEN = math.sqrt(math.pi * 1.0 / N)
TILE = 256
BIG = 1e30


def _bf(x):
    return x.astype(jnp.bfloat16).astype(jnp.float32)


def _first_index_of(vals, target, axis):
    it = jax.lax.broadcasted_iota(jnp.int32, vals.shape, axis)
    return jnp.min(jnp.where(vals == target, it, N), axis=axis, keepdims=True)


def _fps_kernel(px_ref, py_ref, pz_ref, cx_ref, cy_ref, cz_ref):
    px = px_ref[...]
    py = py_ref[...]
    pz = pz_ref[...]
    lane = jax.lax.broadcasted_iota(jnp.int32, (B, N), 1)
    clane = jax.lax.broadcasted_iota(jnp.int32, (B, NPOINT), 1)
    cx_ref[...] = jnp.zeros((B, NPOINT), jnp.float32)
    cy_ref[...] = jnp.zeros((B, NPOINT), jnp.float32)
    cz_ref[...] = jnp.zeros((B, NPOINT), jnp.float32)

    def body(i, state):
        dists, far = state
        eq = lane == far
        cx = jnp.sum(jnp.where(eq, px, 0.0), axis=1, keepdims=True)
        cy = jnp.sum(jnp.where(eq, py, 0.0), axis=1, keepdims=True)
        cz = jnp.sum(jnp.where(eq, pz, 0.0), axis=1, keepdims=True)
        hit = clane == i
        cx_ref[...] = jnp.where(hit, cx, cx_ref[...])
        cy_ref[...] = jnp.where(hit, cy, cy_ref[...])
        cz_ref[...] = jnp.where(hit, cz, cz_ref[...])
        dx = px - cx
        dy = py - cy
        dz = pz - cz
        d = (dx * dx + dy * dy) + dz * dz
        dists = jnp.minimum(dists, d)
        mx = jnp.max(dists, axis=1, keepdims=True)
        far = _first_index_of(dists, mx, 1)
        return dists, far

    dists0 = jnp.full((B, N), 1e10, jnp.float32)
    far0 = jnp.zeros((B, 1), jnp.int32)
    jax.lax.fori_loop(0, NPOINT, body, (dists0, far0))


def _emd_rep_kernel(pxc_ref, pyc_ref, pzc_ref,
                    pxr_ref, pyr_ref, pzr_ref,
                    gxr_ref, gyr_ref, gzr_ref,
                    emd_ref, rep_ref):
    ax = pxc_ref[0]
    ay = pyc_ref[0]
    az = pzc_ref[0]
    an = (ax * ax + ay * ay) + az * az
    axb = _bf(ax)
    ayb = _bf(ay)
    azb = _bf(az)
    lane = jax.lax.broadcasted_iota(jnp.int32, (TILE, N), 1)

    gx = gxr_ref[0]
    gy = gyr_ref[0]
    gz = gzr_ref[0]
    gn = (gx * gx + gy * gy) + gz * gz
    dot = (axb * _bf(gx) + ayb * _bf(gy)) + azb * _bf(gz)
    d2g = jnp.maximum(an + gn - 2.0 * dot, 0.0)
    m = jnp.min(d2g, axis=1, keepdims=True)
    idx = _first_index_of(d2g, m, 1)
    eq = lane == idx
    sgx = jnp.sum(jnp.where(eq, gx, 0.0), axis=1, keepdims=True)
    sgy = jnp.sum(jnp.where(eq, gy, 0.0), axis=1, keepdims=True)
    sgz = jnp.sum(jnp.where(eq, gz, 0.0), axis=1, keepdims=True)
    ex = ax - sgx
    ey = ay - sgy
    ez = az - sgz
    dist2 = (ex * ex + ey * ey) + ez * ez
    emd_ref[0, 0] = jnp.sum(dist2, axis=0, keepdims=True)

    px = pxr_ref[0]
    py = pyr_ref[0]
    pz = pzr_ref[0]
    pn = (px * px + py * py) + pz * pz
    dotp = (axb * _bf(px) + ayb * _bf(py)) + azb * _bf(pz)
    d2p = jnp.maximum(an + pn - 2.0 * dotp, 0.0)
    h = 0.0005
    acc = jnp.zeros((TILE, 1), jnp.float32)
    for r in range(5):
        mr = jnp.min(d2p, axis=1, keepdims=True)
        if r > 0:
            acc = acc + jnp.maximum(h - mr, 0.0)
        if r < 4:
            idxr = _first_index_of(d2p, mr, 1)
            d2p = jnp.where(lane == idxr, BIG, d2p)
    rep_ref[0, 0] = jnp.sum(acc, axis=0, keepdims=True)


def _roll0(x, k):
    if k == 0:
        return x
    return jnp.concatenate([x[k:, :], x[:k, :]], axis=0)


def _uniform_kernel(pxc_ref, pyc_ref, pzc_ref,
                    cxr_ref, cyr_ref, czr_ref, u_ref):
    px = pxc_ref[0]
    py = pyc_ref[0]
    pz = pzc_ref[0]
    pn = (px * px + py * py) + pz * pz
    pxb = _bf(px)
    pyb = _bf(py)
    pzb = _bf(pz)
    cx = cxr_ref[0]
    cy = cyr_ref[0]
    cz = czr_ref[0]
    cn = (cx * cx + cy * cy) + cz * cz
    dot = (pxb * _bf(cx) + pyb * _bf(cy)) + pzb * _bf(cz)
    d2c = jnp.maximum(pn + cn - 2.0 * dot, 0.0)

    for pi, (p, ns) in enumerate(zip(PCTS, NSAMPLES)):
        r2 = p
        mask = d2c < r2
        rank = mask.astype(jnp.int32)
        sh = 1
        while sh < N:
            shifted = jnp.concatenate(
                [jnp.zeros((sh, NPOINT), jnp.int32), rank[: N - sh, :]], axis=0)
            rank = rank + shifted
            sh *= 2
        m = rank[N - 1 :, :]
        gxs, gys, gzs = [], [], []
        for t in range(ns):
            target = jnp.where(t + 1 <= m, t + 1, 1)
            sel = mask & (rank == target)
            vx = jnp.sum(jnp.where(sel, px, 0.0), axis=0, keepdims=True)
            vy = jnp.sum(jnp.where(sel, py, 0.0), axis=0, keepdims=True)
            vz = jnp.sum(jnp.where(sel, pz, 0.0), axis=0, keepdims=True)
            gxs.append(jnp.where(m > 0, vx, px[0:1, :]))
            gys.append(jnp.where(m > 0, vy, py[0:1, :]))
            gzs.append(jnp.where(m > 0, vz, pz[0:1, :]))
        gX = jnp.concatenate(gxs, axis=0)
        gY = jnp.concatenate(gys, axis=0)
        gZ = jnp.concatenate(gzs, axis=0)
        sg = (gX * gX + gY * gY) + gZ * gZ
        gXb = _bf(gX)
        gYb = _bf(gY)
        gZb = _bf(gZ)
        m1 = jnp.full((ns, NPOINT), BIG, jnp.float32)
        m2 = jnp.full((ns, NPOINT), BIG, jnp.float32)
        for k in range(ns):
            dk = (gXb * _roll0(gXb, k) + gYb * _roll0(gYb, k)) \
                + gZb * _roll0(gZb, k)
            v = jnp.maximum(sg + _roll0(sg, k) - 2.0 * dk, 0.0)
            nm1 = jnp.minimum(m1, v)
            m2 = jnp.minimum(m2, jnp.maximum(m1, v))
            m1 = nm1
        dist = jnp.sqrt(jnp.maximum(m2, 1e-12)) + 1e-8
        u = jnp.sum(dist, axis=0, keepdims=True) / float(ns)
        el = EXPECT_LEN
        term = (u - el) * (u - el) / (el + 1e-8)
        u_ref[0, :, pi : pi + 1] = jnp.sum(term, axis=1, keepdims=True)


def kernel(pred, gt, pcd_radius):
    px = pred[:, :, 0]
    py = pred[:, :, 1]
    pz = pred[:, :, 2]

    cx, cy, cz = pl.pallas_call(
        _fps_kernel,
        out_shape=[jax.ShapeDtypeStruct((B, NPOINT), jnp.float32)] * 3,
    )(px, py, pz)

    pxc = px[:, :, None]
    pyc = py[:, :, None]
    pzc = pz[:, :, None]
    pxr = px[:, None, :]
    pyr = py[:, None, :]
    pzr = pz[:, None, :]
    gxr = gt[:, None, :, 0]
    gyr = gt[:, None, :, 1]
    gzr = gt[:, None, :, 2]

    col_spec = pl.BlockSpec((1, TILE, 1), lambda b, t: (b, t, 0))
    row_spec = pl.BlockSpec((1, 1, N), lambda b, t: (b, 0, 0))
    part_spec = pl.BlockSpec((1, 1, 1, 1), lambda b, t: (b, t, 0, 0))
    emd_parts, rep_parts = pl.pallas_call(
        _emd_rep_kernel,
        grid=(B, N // TILE),
        in_specs=[col_spec] * 3 + [row_spec] * 6,
        out_specs=[part_spec, part_spec],
        out_shape=[jax.ShapeDtypeStruct((B, N // TILE, 1, 1), jnp.float32)] * 2,
    )(pxc, pyc, pzc, pxr, pyr, pzr, gxr, gyr, gzr)

    cxr = cx[:, None, :]
    cyr = cy[:, None, :]
    czr = cz[:, None, :]
    pcol_spec = pl.BlockSpec((1, N, 1), lambda b: (b, 0, 0))
    crow_spec = pl.BlockSpec((1, 1, NPOINT), lambda b: (b, 0, 0))
    u_parts = pl.pallas_call(
        _uniform_kernel,
        grid=(B,),
        in_specs=[pcol_spec] * 3 + [crow_spec] * 3,
        out_specs=pl.BlockSpec((1, 1, len(PCTS)), lambda b: (b, 0, 0)),
        out_shape=jax.ShapeDtypeStruct((B, 1, len(PCTS)), jnp.float32),
    )(pxc, pyc, pzc, cxr, cyr, czr)

    emd = jnp.mean((jnp.sum(emd_parts[:, :, 0, 0], axis=1) / (N * 3))[:, None]
                   / pcd_radius) * 100.0

    uni = jnp.float32(0.0)
    for pi, p in enumerate(PCTS):
        loss_p = jnp.sum(u_parts[:, 0, pi]) / (NPOINT * B) * ((p * 100.0) ** 2)
        uni = uni + loss_p
    uni = uni / len(PCTS) * 10.0

    rep = jnp.sum(rep_parts[:, :, 0, 0]) / (B * N * 4) * 5.0
    return (emd, uni, rep)
